# baseline (speedup 1.0000x reference)
import functools

import jax
import jax.numpy as jnp
from jax import lax
from jax.experimental import pallas as pl
from jax.experimental.pallas import tpu as pltpu
from jax.experimental.pallas import tpu_sc as plsc

HIDDEN = 1024
KEY_DIM = 64
MAX_MEM = 16384
T_LEN = 8
K = 8
VD = T_LEN * HIDDEN
NS = 16
L = 16
ROWS_PER_SUB = MAX_MEM // NS
GROUPS = ROWS_PER_SUB // L
W_PER_SUB = KEY_DIM // NS
HCH = HIDDEN // L

_MESH = plsc.VectorSubcoreMesh(core_axis_name="c", subcore_axis_name="s")


@functools.partial(
    pl.kernel,
    out_type=jax.ShapeDtypeStruct((K, VD), jnp.float32),
    mesh=_MESH,
    compiler_params=pltpu.CompilerParams(needs_layout_passes=False),
    scratch_types=[
        pltpu.VMEM((HIDDEN,), jnp.float32),
        pltpu.VMEM((W_PER_SUB * HIDDEN,), jnp.float32),
        pltpu.VMEM((ROWS_PER_SUB * KEY_DIM,), jnp.float32),
        pltpu.VMEM((L,), jnp.float32),
        pltpu.VMEM((NS * L,), jnp.float32),
        pltpu.VMEM((KEY_DIM * L,), jnp.float32),
        pltpu.VMEM((NS * L,), jnp.float32),
        pltpu.VMEM((NS * L,), jnp.int32),
        pltpu.VMEM((L,), jnp.int32),
        pltpu.VMEM((8 * L,), jnp.int32),
        pltpu.VMEM((1, VD), jnp.float32),
        pltpu.VMEM_SHARED((NS * L,), jnp.float32),
        pltpu.VMEM_SHARED((NS * L,), jnp.float32),
        pltpu.VMEM_SHARED((NS * L,), jnp.int32),
        pltpu.SemaphoreType.DMA,
        pltpu.SemaphoreType.DMA,
    ],
)
def _retrieve(q_hbm, keys_hbm, vals_hbm, w_hbm, out_hbm,
              q_v, w_v, keys_v, tmpf_v, qkm_v, bq_v, candv_v, candi_v,
              tmpi_v, idx8_v, row_v, qk_spmem, candv_spmem, candi_spmem,
              sem_keys, sem_row):
    cid = lax.axis_index("c")
    sid = lax.axis_index("s")
    lanes = lax.iota(jnp.int32, L)

    keys_cp = pltpu.async_copy(
        keys_hbm.at[pl.ds(sid * (ROWS_PER_SUB * KEY_DIM), ROWS_PER_SUB * KEY_DIM)],
        keys_v, sem_keys)
    pltpu.sync_copy(q_hbm, q_v)
    pltpu.sync_copy(w_hbm.at[pl.ds(sid * (W_PER_SUB * HIDDEN), W_PER_SUB * HIDDEN)],
                    w_v)

    myvec = jnp.zeros((L,), jnp.float32)
    for j in range(W_PER_SUB):
        def dot_body(h, acc, j=j):
            wv = w_v[pl.ds(j * HIDDEN + h * L, L)]
            qv = q_v[pl.ds(h * L, L)]
            return acc + wv * qv
        acc = lax.fori_loop(0, HCH, dot_body, jnp.zeros((L,), jnp.float32))
        for s in (8, 4, 2, 1):
            tmpf_v[...] = acc
            acc = acc + plsc.load_gather(tmpf_v, [lanes ^ s])
        myvec = jnp.where(lanes == j, acc, myvec)
    tmpf_v[...] = myvec
    pltpu.sync_copy(tmpf_v, qk_spmem.at[pl.ds(sid * L, L)])
    plsc.subcore_barrier()
    pltpu.sync_copy(qk_spmem, qkm_v)

    def bq_body(d, _):
        qpos = (d // W_PER_SUB) * L + (d % W_PER_SUB)
        bv = plsc.load_gather(qkm_v, [jnp.full((L,), 1, jnp.int32) * qpos])
        bq_v[pl.ds(d * L, L)] = bv
        return 0
    lax.fori_loop(0, KEY_DIM, bq_body, 0)

    keys_cp.wait()

    def g_body(g, carry):
        rv, ri = carry
        def d_body(d, acc):
            kidx = g * (L * KEY_DIM) + lanes * KEY_DIM + d
            kv = plsc.load_gather(keys_v, [kidx])
            bv = bq_v[pl.ds(d * L, L)]
            return acc + kv * bv
        acc = lax.fori_loop(0, KEY_DIM, d_body, jnp.zeros((L,), jnp.float32))
        gidx = sid * ROWS_PER_SUB + g * L + lanes
        sv, si = plsc.sort_key_val(acc, gidx, descending=True)
        take = sv > rv
        hv = jnp.where(take, sv, rv)
        hi = jnp.where(take, si, ri)
        rv, ri = plsc.sort_key_val(hv, hi, descending=False)
        return (rv, ri)

    neg_inf = jnp.full((L,), -jnp.inf, jnp.float32)
    rv, ri = lax.fori_loop(0, GROUPS, g_body,
                           (neg_inf, jnp.zeros((L,), jnp.int32)))

    tmpf_v[...] = rv
    tmpi_v[...] = ri
    pltpu.sync_copy(tmpf_v, candv_spmem.at[pl.ds(sid * L, L)])
    pltpu.sync_copy(tmpi_v, candi_spmem.at[pl.ds(sid * L, L)])
    plsc.subcore_barrier()

    @pl.when(sid < K // 2)
    def _():
        pltpu.sync_copy(candv_spmem, candv_v)
        pltpu.sync_copy(candi_spmem, candi_v)

        def m_body(t, carry):
            mrv, mri = carry
            sv = jnp.flip(candv_v[pl.ds(t * L, L)], 0)
            si = jnp.flip(candi_v[pl.ds(t * L, L)], 0)
            take = sv > mrv
            hv = jnp.where(take, sv, mrv)
            hi = jnp.where(take, si, mri)
            mrv, mri = plsc.sort_key_val(hv, hi, descending=False)
            return (mrv, mri)

        mrv, mri = lax.fori_loop(0, NS, m_body,
                                 (neg_inf, jnp.zeros((L,), jnp.int32)))
        best = jnp.flip(mri, 0)
        plsc.store_scatter(idx8_v, [lanes * 8], best)

        k_out = cid * (K // 2) + sid
        pltpu.async_copy(vals_hbm.at[idx8_v.at[pl.ds(k_out * 8, 1)]],
                         row_v, sem_row).wait()
        pltpu.sync_copy(row_v, out_hbm.at[pl.ds(k_out, 1)])


def kernel(query_hidden, keys, values, W_key, top_k):
    del top_k
    out = _retrieve(query_hidden,
                    keys.reshape(-1),
                    values.reshape(MAX_MEM, VD),
                    W_key.reshape(-1))
    return out.reshape(K, T_LEN, HIDDEN)
